# baseline (speedup 1.0000x reference)
import dataclasses
import functools

import jax
import jax.numpy as jnp
from jax import lax
from jax.experimental import pallas as pl
from jax.experimental.pallas import tpu as pltpu
from jax.experimental.pallas import tpu_sc as plsc

_LANES = 16


def _sc_compiler_params():
    cp = pltpu.CompilerParams()
    if "needs_layout_passes" in pltpu.CompilerParams.__dataclass_fields__:
        cp = dataclasses.replace(cp, needs_layout_passes=False)
    return cp


def _gather_coeffs_sc(t, tab_a, tab_s):
    B = t.shape[0]
    T = tab_a.shape[0]
    n_workers = B // _LANES
    mesh = plsc.VectorSubcoreMesh(core_axis_name="c", subcore_axis_name="s")
    num_cores = mesh.num_cores

    @functools.partial(
        pl.kernel,
        out_type=(
            jax.ShapeDtypeStruct((B,), jnp.float32),
            jax.ShapeDtypeStruct((B,), jnp.float32),
        ),
        mesh=mesh,
        scratch_types=[
            pltpu.VMEM((_LANES,), jnp.int32),
            pltpu.VMEM((T,), jnp.float32),
            pltpu.VMEM((T,), jnp.float32),
            pltpu.VMEM((_LANES,), jnp.float32),
            pltpu.VMEM((_LANES,), jnp.float32),
        ],
        compiler_params=_sc_compiler_params(),
    )
    def gather_kernel(t_hbm, ta_hbm, ts_hbm, oa_hbm, os_hbm,
                      idx_v, ta_v, ts_v, va_v, vs_v):
        wid = lax.axis_index("s") * num_cores + lax.axis_index("c")

        @pl.when(wid < n_workers)
        def _():
            base = wid * _LANES
            pltpu.sync_copy(t_hbm.at[pl.ds(base, _LANES)], idx_v)
            pltpu.sync_copy(ta_hbm, ta_v)
            pltpu.sync_copy(ts_hbm, ts_v)
            idx = idx_v[...]
            va_v[...] = plsc.load_gather(ta_v, [idx])
            vs_v[...] = plsc.load_gather(ts_v, [idx])
            pltpu.sync_copy(va_v, oa_hbm.at[pl.ds(base, _LANES)])
            pltpu.sync_copy(vs_v, os_hbm.at[pl.ds(base, _LANES)])

    return gather_kernel(t, tab_a, tab_s)


def _combine_body(a_ref, s_ref, x_ref, n_ref, o_ref):
    o_ref[...] = a_ref[...] * x_ref[...] + s_ref[...] * n_ref[...]


def _combine_tc(x2, n2, a2, s2, rows_per_block):
    B, D = x2.shape
    R = rows_per_block
    return pl.pallas_call(
        _combine_body,
        grid=(B // R,),
        in_specs=[
            pl.BlockSpec((R, 1), lambda i: (i, 0)),
            pl.BlockSpec((R, 1), lambda i: (i, 0)),
            pl.BlockSpec((R, D), lambda i: (i, 0)),
            pl.BlockSpec((R, D), lambda i: (i, 0)),
        ],
        out_specs=pl.BlockSpec((R, D), lambda i: (i, 0)),
        out_shape=jax.ShapeDtypeStruct((B, D), jnp.float32),
        compiler_params=pltpu.CompilerParams(
            dimension_semantics=("arbitrary",),
        ),
    )(a2, s2, x2, n2)


def kernel(x0, t, noise, sqrt_alphas_bar, sqrt_one_minus_alphas_bar):
    B = x0.shape[0]
    D = x0.size // B
    a, s = _gather_coeffs_sc(t, sqrt_alphas_bar, sqrt_one_minus_alphas_bar)
    x2 = x0.reshape(B, D)
    n2 = noise.reshape(B, D)
    out2 = _combine_tc(x2, n2, a.reshape(B, 1), s.reshape(B, 1),
                       rows_per_block=32)
    return out2.reshape(x0.shape)
